# Initial kernel scaffold; baseline (speedup 1.0000x reference)
import jax
import jax.numpy as jnp
from jax import lax
from jax.experimental import pallas as pl
from jax.experimental.pallas import tpu as pltpu
from jax.experimental.pallas import tpu_sc as plsc

N_ROIS = 100000
N_REL = 3200000
SCALE = 28.0 / 1024.0

NC = 2
NS = 16
NW = NC * NS
PER_W = N_REL // NW
CHUNK = 128
N_FULL = PER_W // CHUNK
TAIL = PER_W - N_FULL * CHUNK


def _lanes(i):
  return lax.iota(jnp.int32, 16) + jnp.int32(i * 16)


def _splat(v):
  return jnp.full((16,), v, jnp.int32)


def _process(rois_hbm, rel_hbm, subj_hbm, obj_hbm, inds_v, si_v, oi_v, sb_v,
             ob_v, so_v, oo_v, sem1, sem2, e0, n):
  pltpu.sync_copy(rel_hbm.at[pl.ds(e0, n), :], inds_v.at[pl.ds(0, n), :])
  one = _splat(1)
  two = _splat(2)
  for i in range(n // 16):
    lanes = _lanes(i)
    si_v[pl.ds(i * 16, 16)] = plsc.load_gather(inds_v, [lanes, one])
    oi_v[pl.ds(i * 16, 16)] = plsc.load_gather(inds_v, [lanes, two])
  cp1 = pltpu.async_copy(rois_hbm.at[si_v.at[pl.ds(0, n)]],
                         sb_v.at[pl.ds(0, n), :], sem1)
  cp2 = pltpu.async_copy(rois_hbm.at[oi_v.at[pl.ds(0, n)]],
                         ob_v.at[pl.ds(0, n), :], sem2)
  cp1.wait()
  cp2.wait()
  k = jnp.full((16,), SCALE, jnp.float32)
  cols = [_splat(c) for c in range(5)]
  for i in range(n // 16):
    lanes = _lanes(i)
    s0 = plsc.load_gather(sb_v, [lanes, cols[0]])
    s1 = plsc.load_gather(sb_v, [lanes, cols[1]])
    s2 = plsc.load_gather(sb_v, [lanes, cols[2]])
    s3 = plsc.load_gather(sb_v, [lanes, cols[3]])
    s4 = plsc.load_gather(sb_v, [lanes, cols[4]])
    o0 = plsc.load_gather(ob_v, [lanes, cols[0]])
    o1 = plsc.load_gather(ob_v, [lanes, cols[1]])
    o2 = plsc.load_gather(ob_v, [lanes, cols[2]])
    o3 = plsc.load_gather(ob_v, [lanes, cols[3]])
    o4 = plsc.load_gather(ob_v, [lanes, cols[4]])
    xmin = jnp.minimum(s1, o1)
    ymin = jnp.minimum(s2, o2)
    plsc.store_scatter(so_v, [lanes, cols[0]], s0)
    plsc.store_scatter(so_v, [lanes, cols[1]], (s1 - xmin) * k)
    plsc.store_scatter(so_v, [lanes, cols[2]], (s2 - ymin) * k)
    plsc.store_scatter(so_v, [lanes, cols[3]], (s3 - xmin) * k)
    plsc.store_scatter(so_v, [lanes, cols[4]], (s4 - ymin) * k)
    plsc.store_scatter(oo_v, [lanes, cols[0]], o0)
    plsc.store_scatter(oo_v, [lanes, cols[1]], (o1 - xmin) * k)
    plsc.store_scatter(oo_v, [lanes, cols[2]], (o2 - ymin) * k)
    plsc.store_scatter(oo_v, [lanes, cols[3]], (o3 - xmin) * k)
    plsc.store_scatter(oo_v, [lanes, cols[4]], (o4 - ymin) * k)
  pltpu.sync_copy(so_v.at[pl.ds(0, n), :], subj_hbm.at[pl.ds(e0, n), :])
  pltpu.sync_copy(oo_v.at[pl.ds(0, n), :], obj_hbm.at[pl.ds(e0, n), :])


def _sc_body(rois_hbm, rel_hbm, subj_hbm, obj_hbm, inds_v, si_v, oi_v, sb_v,
             ob_v, so_v, oo_v, sem1, sem2):
  wid = lax.axis_index("s") * NC + lax.axis_index("c")
  base = wid * PER_W

  def chunk_body(j, carry):
    _process(rois_hbm, rel_hbm, subj_hbm, obj_hbm, inds_v, si_v, oi_v, sb_v,
             ob_v, so_v, oo_v, sem1, sem2, base + j * CHUNK, CHUNK)
    return carry

  lax.fori_loop(0, N_FULL, chunk_body, jnp.int32(0))
  if TAIL:
    _process(rois_hbm, rel_hbm, subj_hbm, obj_hbm, inds_v, si_v, oi_v, sb_v,
             ob_v, so_v, oo_v, sem1, sem2, base + N_FULL * CHUNK, TAIL)


@jax.jit
def kernel(rois, rel_inds):
  rois_pad = jnp.pad(rois, ((0, 0), (0, 3)))
  mesh = plsc.VectorSubcoreMesh(core_axis_name="c", subcore_axis_name="s")
  f = pl.kernel(
      _sc_body,
      out_type=(
          jax.ShapeDtypeStruct((N_REL, 5), jnp.float32),
          jax.ShapeDtypeStruct((N_REL, 5), jnp.float32),
      ),
      mesh=mesh,
      scratch_types=[
          pltpu.VMEM((CHUNK, 3), jnp.int32),
          pltpu.VMEM((CHUNK,), jnp.int32),
          pltpu.VMEM((CHUNK,), jnp.int32),
          pltpu.VMEM((CHUNK, 8), jnp.float32),
          pltpu.VMEM((CHUNK, 8), jnp.float32),
          pltpu.VMEM((CHUNK, 5), jnp.float32),
          pltpu.VMEM((CHUNK, 5), jnp.float32),
          pltpu.SemaphoreType.DMA,
          pltpu.SemaphoreType.DMA,
      ],
  )
  return f(rois_pad, rel_inds)


# speedup vs baseline: 1.6744x; 1.6744x over previous
import jax
import jax.numpy as jnp
from jax import lax
from jax.experimental import pallas as pl
from jax.experimental.pallas import tpu as pltpu
from jax.experimental.pallas import tpu_sc as plsc

N_ROIS = 100000
N_REL = 3200000
SCALE = 28.0 / 1024.0

NC = 2
NS = 16
NW = NC * NS
PER_W = N_REL // NW
CHUNK = 128
N_FULL = PER_W // CHUNK
TAIL = PER_W - N_FULL * CHUNK


def _lanes(i):
  return lax.iota(jnp.int32, 16) + jnp.int32(i * 16)


def _splat(v):
  return jnp.full((16,), v, jnp.int32)


def _process(rois_hbm, rel_hbm, subj_hbm, obj_hbm, inds_v, si_v, oi_v, sb_v,
             ob_v, so_v, oo_v, sem1, sem2, e0, n):
  pltpu.sync_copy(rel_hbm.at[pl.ds(e0, n), :], inds_v.at[pl.ds(0, n), :])
  one = _splat(1)
  two = _splat(2)
  for i in range(n // 16):
    lanes = _lanes(i)
    si_v[pl.ds(i * 16, 16)] = plsc.load_gather(inds_v, [lanes, one])
    oi_v[pl.ds(i * 16, 16)] = plsc.load_gather(inds_v, [lanes, two])
  cp1 = pltpu.async_copy(rois_hbm.at[si_v.at[pl.ds(0, n)]],
                         sb_v.at[pl.ds(0, n), :], sem1)
  cp2 = pltpu.async_copy(rois_hbm.at[oi_v.at[pl.ds(0, n)]],
                         ob_v.at[pl.ds(0, n), :], sem2)
  cp1.wait()
  cp2.wait()
  k = jnp.full((16,), SCALE, jnp.float32)
  cols = [_splat(c) for c in range(5)]
  for i in range(n // 16):
    lanes = _lanes(i)
    s0 = plsc.load_gather(sb_v, [lanes, cols[0]])
    s1 = plsc.load_gather(sb_v, [lanes, cols[1]])
    s2 = plsc.load_gather(sb_v, [lanes, cols[2]])
    s3 = plsc.load_gather(sb_v, [lanes, cols[3]])
    s4 = plsc.load_gather(sb_v, [lanes, cols[4]])
    o0 = plsc.load_gather(ob_v, [lanes, cols[0]])
    o1 = plsc.load_gather(ob_v, [lanes, cols[1]])
    o2 = plsc.load_gather(ob_v, [lanes, cols[2]])
    o3 = plsc.load_gather(ob_v, [lanes, cols[3]])
    o4 = plsc.load_gather(ob_v, [lanes, cols[4]])
    xmin = jnp.minimum(s1, o1)
    ymin = jnp.minimum(s2, o2)
    plsc.store_scatter(so_v, [lanes, cols[0]], s0)
    plsc.store_scatter(so_v, [lanes, cols[1]], (s1 - xmin) * k)
    plsc.store_scatter(so_v, [lanes, cols[2]], (s2 - ymin) * k)
    plsc.store_scatter(so_v, [lanes, cols[3]], (s3 - xmin) * k)
    plsc.store_scatter(so_v, [lanes, cols[4]], (s4 - ymin) * k)
    plsc.store_scatter(oo_v, [lanes, cols[0]], o0)
    plsc.store_scatter(oo_v, [lanes, cols[1]], (o1 - xmin) * k)
    plsc.store_scatter(oo_v, [lanes, cols[2]], (o2 - ymin) * k)
    plsc.store_scatter(oo_v, [lanes, cols[3]], (o3 - xmin) * k)
    plsc.store_scatter(oo_v, [lanes, cols[4]], (o4 - ymin) * k)
  pltpu.sync_copy(so_v.at[pl.ds(0, n), :], subj_hbm.at[pl.ds(e0, n), :])
  pltpu.sync_copy(oo_v.at[pl.ds(0, n), :], obj_hbm.at[pl.ds(e0, n), :])


def _sc_body(rois_hbm, rel_hbm, subj_hbm, obj_hbm, inds_v, si_v, oi_v, sb_v,
             ob_v, so_v, oo_v, sem1, sem2):
  wid = lax.axis_index("s") * NC + lax.axis_index("c")
  base = wid * PER_W

  def chunk_body(j, carry):
    _process(rois_hbm, rel_hbm, subj_hbm, obj_hbm, inds_v, si_v, oi_v, sb_v,
             ob_v, so_v, oo_v, sem1, sem2, base + j * CHUNK, CHUNK)
    return carry

  lax.fori_loop(0, N_FULL, chunk_body, jnp.int32(0))
  if TAIL:
    _process(rois_hbm, rel_hbm, subj_hbm, obj_hbm, inds_v, si_v, oi_v, sb_v,
             ob_v, so_v, oo_v, sem1, sem2, base + N_FULL * CHUNK, TAIL)


@jax.jit
def kernel(rois, rel_inds):
  rois_pad = jnp.pad(rois, ((0, 0), (0, 3)))
  mesh = plsc.VectorSubcoreMesh(core_axis_name="c", subcore_axis_name="s")
  f = pl.kernel(
      _sc_body,
      out_type=(
          jax.ShapeDtypeStruct((N_REL, 5), jnp.float32),
          jax.ShapeDtypeStruct((N_REL, 5), jnp.float32),
      ),
      mesh=mesh,
      compiler_params=pltpu.CompilerParams(
          needs_layout_passes=False, use_tc_tiling_on_sc=False),
      scratch_types=[
          pltpu.VMEM((CHUNK, 3), jnp.int32),
          pltpu.VMEM((CHUNK,), jnp.int32),
          pltpu.VMEM((CHUNK,), jnp.int32),
          pltpu.VMEM((CHUNK, 8), jnp.float32),
          pltpu.VMEM((CHUNK, 8), jnp.float32),
          pltpu.VMEM((CHUNK, 5), jnp.float32),
          pltpu.VMEM((CHUNK, 5), jnp.float32),
          pltpu.SemaphoreType.DMA,
          pltpu.SemaphoreType.DMA,
      ],
  )
  return f(rois_pad, rel_inds)


# speedup vs baseline: 1.7987x; 1.0742x over previous
import jax
import jax.numpy as jnp
from jax import lax
from jax.experimental import pallas as pl
from jax.experimental.pallas import tpu as pltpu
from jax.experimental.pallas import tpu_sc as plsc

N_ROIS = 100000
N_REL = 3200000
SCALE = 28.0 / 1024.0

NC = 2
NS = 16
NW = NC * NS
PER_W = N_REL // NW
CHUNK = 400
SUB = 80
NSUB = CHUNK // SUB
NCH = PER_W // CHUNK


def _iota16():
  return lax.iota(jnp.int32, 16)


def _splat(v):
  return jnp.full((16,), v, jnp.int32)


def _sc_body(rois_hbm, rel_hbm, subj_hbm, obj_hbm,
             inds0, inds1, si0, si1, oi0, oi1, sb0, sb1, ob0, ob1,
             so0, so1, oo0, oo1, semi0, semi1, semg0, semg1, semo0, semo1):
  inds = (inds0, inds1)
  si = (si0, si1)
  oi = (oi0, oi1)
  sb = (sb0, sb1)
  ob = (ob0, ob1)
  so = (so0, so1)
  oo = (oo0, oo1)
  semi = (semi0, semi1)
  semg = (semg0, semg1)
  semo = (semo0, semo1)

  wid = lax.axis_index("s") * NC + lax.axis_index("c")
  base = wid * PER_W

  def fire_inds(b, c):
    pltpu.async_copy(rel_hbm.at[pl.ds(base + c * CHUNK, CHUNK), :], inds[b],
                     semi[b])

  def wait_inds(b):
    pltpu.make_async_copy(rel_hbm.at[pl.ds(0, CHUNK), :], inds[b],
                          semi[b]).wait()

  def extract(b):
    one = _splat(1)
    two = _splat(2)
    it = _iota16()
    for i in range(CHUNK // 16):
      lanes = it + _splat(i * 16)
      row = _splat(i // NSUB)
      col = it + _splat((i % NSUB) * 16)
      s = plsc.load_gather(inds[b], [lanes, one])
      o = plsc.load_gather(inds[b], [lanes, two])
      plsc.store_scatter(si[b], [row, col], s)
      plsc.store_scatter(oi[b], [row, col], o)

  def fire_gathers(b):
    for g in range(NSUB):
      pltpu.async_copy(rois_hbm.at[si[b].at[g]],
                       sb[b].at[pl.ds(g * SUB, SUB), :], semg[b])
      pltpu.async_copy(rois_hbm.at[oi[b].at[g]],
                       ob[b].at[pl.ds(g * SUB, SUB), :], semg[b])

  def wait_gathers(b):
    pltpu.make_async_copy(rois_hbm.at[pl.ds(0, CHUNK), :], sb[b],
                          semg[b]).wait()
    pltpu.make_async_copy(rois_hbm.at[pl.ds(0, CHUNK), :], ob[b],
                          semg[b]).wait()

  def compute(b):
    k = jnp.full((16,), SCALE, jnp.float32)
    cols = [_splat(c) for c in range(5)]
    it = _iota16()
    for i in range(CHUNK // 16):
      lanes = it + _splat(i * 16)
      s0 = plsc.load_gather(sb[b], [lanes, cols[0]])
      s1 = plsc.load_gather(sb[b], [lanes, cols[1]])
      s2 = plsc.load_gather(sb[b], [lanes, cols[2]])
      s3 = plsc.load_gather(sb[b], [lanes, cols[3]])
      s4 = plsc.load_gather(sb[b], [lanes, cols[4]])
      o0 = plsc.load_gather(ob[b], [lanes, cols[0]])
      o1 = plsc.load_gather(ob[b], [lanes, cols[1]])
      o2 = plsc.load_gather(ob[b], [lanes, cols[2]])
      o3 = plsc.load_gather(ob[b], [lanes, cols[3]])
      o4 = plsc.load_gather(ob[b], [lanes, cols[4]])
      xmin = jnp.minimum(s1, o1)
      ymin = jnp.minimum(s2, o2)
      plsc.store_scatter(so[b], [lanes, cols[0]], s0)
      plsc.store_scatter(so[b], [lanes, cols[1]], (s1 - xmin) * k)
      plsc.store_scatter(so[b], [lanes, cols[2]], (s2 - ymin) * k)
      plsc.store_scatter(so[b], [lanes, cols[3]], (s3 - xmin) * k)
      plsc.store_scatter(so[b], [lanes, cols[4]], (s4 - ymin) * k)
      plsc.store_scatter(oo[b], [lanes, cols[0]], o0)
      plsc.store_scatter(oo[b], [lanes, cols[1]], (o1 - xmin) * k)
      plsc.store_scatter(oo[b], [lanes, cols[2]], (o2 - ymin) * k)
      plsc.store_scatter(oo[b], [lanes, cols[3]], (o3 - xmin) * k)
      plsc.store_scatter(oo[b], [lanes, cols[4]], (o4 - ymin) * k)

  def fire_out(b, c):
    pltpu.async_copy(so[b], subj_hbm.at[pl.ds(base + c * CHUNK, CHUNK), :],
                     semo[b])
    pltpu.async_copy(oo[b], obj_hbm.at[pl.ds(base + c * CHUNK, CHUNK), :],
                     semo[b])

  def wait_out(b):
    pltpu.make_async_copy(so[b], subj_hbm.at[pl.ds(0, CHUNK), :],
                          semo[b]).wait()
    pltpu.make_async_copy(oo[b], obj_hbm.at[pl.ds(0, CHUNK), :],
                          semo[b]).wait()

  fire_inds(0, jnp.int32(0))
  wait_inds(0)
  extract(0)
  fire_gathers(0)
  fire_inds(1, jnp.int32(1))

  def pair_body(j2, carry):
    for b in (0, 1):
      j = j2 * 2 + b
      nb = 1 - b

      @pl.when(j < NCH - 1)
      def _prep():
        wait_inds(nb)
        extract(nb)
        fire_gathers(nb)

      @pl.when(j < NCH - 2)
      def _pref():
        fire_inds(b, j + 2)

      wait_gathers(b)

      @pl.when(j >= 2)
      def _drain():
        wait_out(b)

      compute(b)
      fire_out(b, j)
    return carry

  lax.fori_loop(0, NCH // 2, pair_body, jnp.int32(0))
  wait_out(0)
  wait_out(1)


@jax.jit
def kernel(rois, rel_inds):
  rois_pad = jnp.pad(rois, ((0, 0), (0, 3)))
  mesh = plsc.VectorSubcoreMesh(core_axis_name="c", subcore_axis_name="s")
  f = pl.kernel(
      _sc_body,
      out_type=(
          jax.ShapeDtypeStruct((N_REL, 5), jnp.float32),
          jax.ShapeDtypeStruct((N_REL, 5), jnp.float32),
      ),
      mesh=mesh,
      compiler_params=pltpu.CompilerParams(
          needs_layout_passes=False, use_tc_tiling_on_sc=False),
      scratch_types=[
          pltpu.VMEM((CHUNK, 3), jnp.int32),
          pltpu.VMEM((CHUNK, 3), jnp.int32),
          pltpu.VMEM((NSUB, SUB), jnp.int32),
          pltpu.VMEM((NSUB, SUB), jnp.int32),
          pltpu.VMEM((NSUB, SUB), jnp.int32),
          pltpu.VMEM((NSUB, SUB), jnp.int32),
          pltpu.VMEM((CHUNK, 8), jnp.float32),
          pltpu.VMEM((CHUNK, 8), jnp.float32),
          pltpu.VMEM((CHUNK, 8), jnp.float32),
          pltpu.VMEM((CHUNK, 8), jnp.float32),
          pltpu.VMEM((CHUNK, 5), jnp.float32),
          pltpu.VMEM((CHUNK, 5), jnp.float32),
          pltpu.VMEM((CHUNK, 5), jnp.float32),
          pltpu.VMEM((CHUNK, 5), jnp.float32),
          pltpu.SemaphoreType.DMA,
          pltpu.SemaphoreType.DMA,
          pltpu.SemaphoreType.DMA,
          pltpu.SemaphoreType.DMA,
          pltpu.SemaphoreType.DMA,
          pltpu.SemaphoreType.DMA,
      ],
  )
  return f(rois_pad, rel_inds)
